# Initial kernel scaffold; baseline (speedup 1.0000x reference)
import jax
import jax.numpy as jnp
from jax.experimental import pallas as pl


def kernel(inputs, embeddings):
    raise NotImplementedError("write your pallas kernel here")


# speedup vs baseline: 1.6998x; 1.6998x over previous
import jax
import jax.numpy as jnp
from jax import lax
from jax.experimental import pallas as pl
from jax.experimental.pallas import tpu as pltpu

_K = 1024
_D = 32
_N = 65536
_R = 2048
_GRID = _N // _R
_CC = 6.0
_EPS = 1e-05


def _vq_body(x_ref, e_ref, q_ref, idx_ref, loss_ref, perp_ref,
             counts_ref, ssd_ref):
    i = pl.program_id(0)

    @pl.when(i == 0)
    def _init():
        counts_ref[...] = jnp.zeros_like(counts_ref)
        ssd_ref[0] = 0.0

    x = x_ref[...]
    e = e_ref[...]
    score = jnp.dot(x, e, preferred_element_type=jnp.float32)
    x2 = jnp.sum(x * x, axis=1, keepdims=True)
    e2 = jnp.sum(e * e, axis=0, keepdims=True)
    dist = (x2 + e2) - 2.0 * score
    idx = jnp.argmin(dist, axis=1).astype(jnp.int32)
    one_hot = (lax.broadcasted_iota(jnp.int32, (_R, _K), 1)
               == idx[:, None]).astype(jnp.float32)
    q = lax.dot_general(one_hot, e, (((1,), (1,)), ((), ())),
                        preferred_element_type=jnp.float32)
    q_ref[...] = q
    idx_ref[...] = idx
    counts_ref[...] += jnp.sum(one_hot, axis=0, keepdims=True)
    d = q - x
    ssd_ref[0] += jnp.sum(d * d)

    @pl.when(i == _GRID - 1)
    def _fin():
        loss_ref[...] = jnp.full((1, 1), (_CC / (_N * _D)) * ssd_ref[0],
                                 jnp.float32)
        p = counts_ref[...] * (1.0 / _N)
        ent = jnp.sum(p * jnp.log(p + _EPS), axis=1, keepdims=True)
        perp_ref[...] = jnp.exp(-ent)


def kernel(inputs, embeddings):
    x = inputs.reshape(_N, _D)
    q, idx, loss, perp = pl.pallas_call(
        _vq_body,
        grid=(_GRID,),
        in_specs=[
            pl.BlockSpec((_R, _D), lambda i: (i, 0)),
            pl.BlockSpec((_D, _K), lambda i: (0, 0)),
        ],
        out_specs=[
            pl.BlockSpec((_R, _D), lambda i: (i, 0)),
            pl.BlockSpec((_R,), lambda i: (i,)),
            pl.BlockSpec((1, 1), lambda i: (0, 0)),
            pl.BlockSpec((1, 1), lambda i: (0, 0)),
        ],
        out_shape=[
            jax.ShapeDtypeStruct((_N, _D), jnp.float32),
            jax.ShapeDtypeStruct((_N,), jnp.int32),
            jax.ShapeDtypeStruct((1, 1), jnp.float32),
            jax.ShapeDtypeStruct((1, 1), jnp.float32),
        ],
        scratch_shapes=[
            pltpu.VMEM((1, _K), jnp.float32),
            pltpu.SMEM((1,), jnp.float32),
        ],
    )(x, embeddings)
    loss = loss.reshape(())
    perp = perp.reshape(())
    quantized = q.reshape(inputs.shape)
    idx_out = idx.reshape(inputs.shape[:-1])
    return (loss, quantized, idx_out, perp)


# speedup vs baseline: 1.7530x; 1.0313x over previous
import jax
import jax.numpy as jnp
from jax import lax
from jax.experimental import pallas as pl
from jax.experimental.pallas import tpu as pltpu

_K = 1024
_D = 32
_N = 65536
_B = 2
_R = _B * 1024
_GRID = _N // _R
_CC = 6.0
_EPS = 1e-05


def _vq_body(x_ref, e_ref, q_ref, idx_ref, loss_ref, perp_ref,
             counts_ref, ssd_ref):
    i = pl.program_id(0)

    @pl.when(i == 0)
    def _init():
        counts_ref[...] = jnp.zeros_like(counts_ref)
        ssd_ref[0] = 0.0

    x = x_ref[...].reshape(_R, _D)
    e = e_ref[...]
    score = jnp.dot(x, e, preferred_element_type=jnp.float32)
    x2 = jnp.sum(x * x, axis=1, keepdims=True)
    e2 = jnp.sum(e * e, axis=0, keepdims=True)
    dist = (x2 + e2) - 2.0 * score
    idx = jnp.argmin(dist, axis=1).astype(jnp.int32)
    one_hot = (lax.broadcasted_iota(jnp.int32, (_R, _K), 1)
               == idx[:, None]).astype(jnp.float32)
    q = lax.dot_general(one_hot, e, (((1,), (1,)), ((), ())),
                        preferred_element_type=jnp.float32)
    q_ref[...] = q.reshape(_B, 32, 32, _D)
    idx_ref[...] = idx.reshape(_B, 32, 32)
    counts_ref[...] += jnp.sum(one_hot, axis=0, keepdims=True)
    d = q - x
    ssd_ref[0] += jnp.sum(d * d)

    @pl.when(i == _GRID - 1)
    def _fin():
        loss_ref[...] = jnp.full((1, 1), (_CC / (_N * _D)) * ssd_ref[0],
                                 jnp.float32)
        p = counts_ref[...] * (1.0 / _N)
        ent = jnp.sum(p * jnp.log(p + _EPS), axis=1, keepdims=True)
        perp_ref[...] = jnp.exp(-ent)


def kernel(inputs, embeddings):
    q, idx, loss, perp = pl.pallas_call(
        _vq_body,
        grid=(_GRID,),
        in_specs=[
            pl.BlockSpec((_B, 32, 32, _D), lambda i: (i, 0, 0, 0)),
            pl.BlockSpec((_D, _K), lambda i: (0, 0)),
        ],
        out_specs=[
            pl.BlockSpec((_B, 32, 32, _D), lambda i: (i, 0, 0, 0)),
            pl.BlockSpec((_B, 32, 32), lambda i: (i, 0, 0)),
            pl.BlockSpec((1, 1), lambda i: (0, 0)),
            pl.BlockSpec((1, 1), lambda i: (0, 0)),
        ],
        out_shape=[
            jax.ShapeDtypeStruct((64, 32, 32, _D), jnp.float32),
            jax.ShapeDtypeStruct((64, 32, 32), jnp.int32),
            jax.ShapeDtypeStruct((1, 1), jnp.float32),
            jax.ShapeDtypeStruct((1, 1), jnp.float32),
        ],
        scratch_shapes=[
            pltpu.VMEM((1, _K), jnp.float32),
            pltpu.SMEM((1,), jnp.float32),
        ],
    )(inputs, embeddings)
    return (loss.reshape(()), q, idx, perp.reshape(()))


# speedup vs baseline: 1.8136x; 1.0345x over previous
import jax
import jax.numpy as jnp
from jax import lax
from jax.experimental import pallas as pl
from jax.experimental.pallas import tpu as pltpu

_K = 1024
_D = 32
_N = 65536
_B = 4
_R = _B * 1024
_GRID = _N // _R
_CC = 6.0
_EPS = 1e-05


def _vq_body(x_ref, e_ref, q_ref, idx_ref, loss_ref, perp_ref,
             counts_ref, ssd_ref):
    i = pl.program_id(0)

    @pl.when(i == 0)
    def _init():
        counts_ref[...] = jnp.zeros_like(counts_ref)
        ssd_ref[0] = 0.0

    x = x_ref[...].reshape(_R, _D)
    e = e_ref[...]
    score = jnp.dot(x, e, preferred_element_type=jnp.float32)
    x2 = jnp.sum(x * x, axis=1, keepdims=True)
    e2 = jnp.sum(e * e, axis=0, keepdims=True)
    dist = (x2 + e2) - 2.0 * score
    idx = jnp.argmin(dist, axis=1).astype(jnp.int32)
    one_hot = (lax.broadcasted_iota(jnp.int32, (_R, _K), 1)
               == idx[:, None]).astype(jnp.float32)
    q = lax.dot_general(one_hot, e, (((1,), (1,)), ((), ())),
                        preferred_element_type=jnp.float32)
    q_ref[...] = q.reshape(_B, 32, 32, _D)
    idx_ref[...] = idx.reshape(_B, 32, 32)
    counts_ref[...] += jnp.sum(one_hot, axis=0, keepdims=True)
    d = q - x
    ssd_ref[0] += jnp.sum(d * d)

    @pl.when(i == _GRID - 1)
    def _fin():
        loss_ref[...] = jnp.full((1, 1), (_CC / (_N * _D)) * ssd_ref[0],
                                 jnp.float32)
        p = counts_ref[...] * (1.0 / _N)
        ent = jnp.sum(p * jnp.log(p + _EPS), axis=1, keepdims=True)
        perp_ref[...] = jnp.exp(-ent)


def kernel(inputs, embeddings):
    q, idx, loss, perp = pl.pallas_call(
        _vq_body,
        grid=(_GRID,),
        in_specs=[
            pl.BlockSpec((_B, 32, 32, _D), lambda i: (i, 0, 0, 0)),
            pl.BlockSpec((_D, _K), lambda i: (0, 0)),
        ],
        out_specs=[
            pl.BlockSpec((_B, 32, 32, _D), lambda i: (i, 0, 0, 0)),
            pl.BlockSpec((_B, 32, 32), lambda i: (i, 0, 0)),
            pl.BlockSpec((1, 1), lambda i: (0, 0)),
            pl.BlockSpec((1, 1), lambda i: (0, 0)),
        ],
        out_shape=[
            jax.ShapeDtypeStruct((64, 32, 32, _D), jnp.float32),
            jax.ShapeDtypeStruct((64, 32, 32), jnp.int32),
            jax.ShapeDtypeStruct((1, 1), jnp.float32),
            jax.ShapeDtypeStruct((1, 1), jnp.float32),
        ],
        scratch_shapes=[
            pltpu.VMEM((1, _K), jnp.float32),
            pltpu.SMEM((1,), jnp.float32),
        ],
    )(inputs, embeddings)
    return (loss.reshape(()), q, idx, perp.reshape(()))
